# baseline (speedup 1.0000x reference)
import functools

import jax
import jax.numpy as jnp
from jax import lax
from jax.experimental import pallas as pl
from jax.experimental.pallas import tpu as pltpu
from jax.experimental.pallas import tpu_sc as plsc

T = 2048
D = 2048
E = 8
DFF = 768
K = 2

BLK = 256
PMAX = T * K + E * BLK
NB = PMAX // BLK

NW = 32
TOK_W = T // NW
CT = 16


def _route_body(x_ref, gw_ref, pos_ref, sc1_ref, sc2_ref, bexp_ref):
    x = x_ref[...]
    logits = lax.dot_general(x, gw_ref[...], (((1,), (1,)), ((), ())),
                             preferred_element_type=jnp.float32)
    probs = jax.nn.softmax(logits, axis=-1)
    iota_e = lax.broadcasted_iota(jnp.int32, probs.shape, 1)
    m1 = jnp.max(probs, axis=1, keepdims=True)
    idx1 = jnp.min(jnp.where(probs == m1, iota_e, E), axis=1, keepdims=True)
    oh1 = iota_e == idx1
    p2 = jnp.where(oh1, -jnp.inf, probs)
    m2 = jnp.max(p2, axis=1, keepdims=True)
    idx2 = jnp.min(jnp.where(p2 == m2, iota_e, E), axis=1, keepdims=True)
    oh2 = iota_e == idx2
    denom = m1 + m2
    s1 = m1 / denom
    s2 = m2 / denom

    scnt = oh1.astype(jnp.float32) + oh2.astype(jnp.float32)
    r_i = lax.broadcasted_iota(jnp.int32, (T, T), 0)
    c_i = lax.broadcasted_iota(jnp.int32, (T, T), 1)
    ltri = (c_i < r_i).astype(jnp.float32)
    csum = lax.dot_general(ltri, scnt, (((1,), (0,)), ((), ())),
                           preferred_element_type=jnp.float32)
    counts = csum[T - 1:T, :] + scnt[T - 1:T, :]
    padded = jnp.floor((counts + (BLK - 1)) * (1.0 / BLK)).astype(jnp.float32)
    padded = padded * BLK
    a_i = lax.broadcasted_iota(jnp.int32, (E, E), 0)
    b_i = lax.broadcasted_iota(jnp.int32, (E, E), 1)
    utri = (a_i < b_i).astype(jnp.float32)
    offs = lax.dot_general(padded, utri, (((1,), (0,)), ((), ())),
                           preferred_element_type=jnp.float32)

    slot = offs + csum
    pos1 = jnp.sum(jnp.where(oh1, slot, 0.0), axis=1, keepdims=True)
    pos2 = jnp.sum(jnp.where(oh2, slot, 0.0), axis=1, keepdims=True)

    rowp = lax.broadcasted_iota(jnp.int32, (8, T), 0)
    pos1_t = jnp.transpose(pos1.astype(jnp.int32))
    pos2_t = jnp.transpose(pos2.astype(jnp.int32))
    pos_ref[...] = jnp.where(rowp == 0, pos1_t,
                             jnp.where(rowp == 1, pos2_t, 0))
    sc1_ref[...] = jnp.broadcast_to(s1, (T, 128))
    sc2_ref[...] = jnp.broadcast_to(s2, (T, 128))

    rowb = lax.broadcasted_iota(jnp.int32, (8, 128), 0)
    laneb = lax.broadcasted_iota(jnp.int32, (8, 128), 1)
    startf = (laneb * BLK).astype(jnp.float32)
    ends = offs + padded
    acc = jnp.zeros((8, 128), jnp.int32)
    for e in range(E - 1):
        acc = acc + (startf >= ends[0:1, e:e + 1]).astype(jnp.int32)
    total = ends[0:1, E - 1:E]
    valid = (startf < total).astype(jnp.int32)
    bexp_ref[...] = jnp.where(rowb == 0, acc, valid)


def _route(x, gw):
    return pl.pallas_call(
        _route_body,
        out_shape=(
            jax.ShapeDtypeStruct((8, T), jnp.int32),
            jax.ShapeDtypeStruct((T, 128), jnp.float32),
            jax.ShapeDtypeStruct((T, 128), jnp.float32),
            jax.ShapeDtypeStruct((8, 128), jnp.int32),
        ),
    )(x, gw)


NCH = TOK_W // CT


def _dispatch_body(x_hbm, p_hbm, s1_hbm, s2_hbm, xs_hbm, ws_hbm, *scr):
    bufs = (scr[0:5], scr[5:10])
    sem_ld = scr[10]
    sem_st = scr[11]
    wid = lax.axis_index("s") * 2 + lax.axis_index("c")
    base = wid * TOK_W

    def fire_loads(ci, bs):
        xbuf, i0, i1, sv0, sv1 = bs
        tb = base + ci * CT
        return [
            pltpu.async_copy(x_hbm.at[pl.ds(tb, CT)], xbuf, sem_ld),
            pltpu.async_copy(p_hbm.at[0, pl.ds(tb, CT)], i0, sem_ld),
            pltpu.async_copy(p_hbm.at[1, pl.ds(tb, CT)], i1, sem_ld),
            pltpu.async_copy(s1_hbm.at[pl.ds(tb, CT)], sv0, sem_ld),
            pltpu.async_copy(s2_hbm.at[pl.ds(tb, CT)], sv1, sem_ld),
        ]

    def fire_scats(bs):
        xbuf, i0, i1, sv0, sv1 = bs
        return [
            pltpu.async_copy(xbuf, xs_hbm.at[i0], sem_st),
            pltpu.async_copy(xbuf, xs_hbm.at[i1], sem_st),
            pltpu.async_copy(sv0, ws_hbm.at[i0], sem_st),
            pltpu.async_copy(sv1, ws_hbm.at[i1], sem_st),
        ]

    lds = fire_loads(0, bufs[0])
    pend = []
    for ci in range(NCH):
        for d in lds:
            d.wait()
        for d in pend:
            d.wait()
        if ci + 1 < NCH:
            lds = fire_loads(ci + 1, bufs[(ci + 1) % 2])
        pend = fire_scats(bufs[ci % 2])
    for d in pend:
        d.wait()


def _dispatch(x, pos, s1, s2):
    mesh = plsc.VectorSubcoreMesh(core_axis_name="c", subcore_axis_name="s", num_cores=2, num_subcores=16)
    bufset = [
        pltpu.VMEM((CT, D), jnp.float32),
        pltpu.VMEM((CT,), jnp.int32),
        pltpu.VMEM((CT,), jnp.int32),
        pltpu.VMEM((CT, 128), jnp.float32),
        pltpu.VMEM((CT, 128), jnp.float32),
    ]
    f = pl.kernel(
        _dispatch_body,
        out_type=(
            jax.ShapeDtypeStruct((PMAX, D), jnp.float32),
            jax.ShapeDtypeStruct((PMAX, 128), jnp.float32),
        ),
        mesh=mesh,
        scratch_types=bufset + bufset + [pltpu.SemaphoreType.DMA,
                                         pltpu.SemaphoreType.DMA],
    )
    return f(x, pos, s1, s2)


def _gmm_body(bexp_ref, xs_ref, ws_ref, w13_ref, w2_ref, ys_ref):
    @pl.when(bexp_ref[1, pl.program_id(0)] == 1)
    def _():
        xs = xs_ref[...]
        g = lax.dot_general(xs, w13_ref[0, :DFF, :], (((1,), (1,)), ((), ())),
                            preferred_element_type=jnp.float32)
        u = lax.dot_general(xs, w13_ref[0, DFF:, :], (((1,), (1,)), ((), ())),
                            preferred_element_type=jnp.float32)
        act = g * jax.nn.sigmoid(g) * u
        y = lax.dot_general(act, w2_ref[0], (((1,), (1,)), ((), ())),
                            preferred_element_type=jnp.float32)
        ys_ref[...] = y * ws_ref[:, 0:1]


def _gmm(bexp, xs, ws, w13, w2):
    grid_spec = pltpu.PrefetchScalarGridSpec(
        num_scalar_prefetch=1,
        grid=(NB,),
        in_specs=[
            pl.BlockSpec((BLK, D), lambda b, be: (b, 0)),
            pl.BlockSpec((BLK, 128), lambda b, be: (b, 0)),
            pl.BlockSpec((1, 2 * DFF, D), lambda b, be: (be[0, b], 0, 0)),
            pl.BlockSpec((1, D, DFF), lambda b, be: (be[0, b], 0, 0)),
        ],
        out_specs=pl.BlockSpec((BLK, D), lambda b, be: (b, 0)),
    )
    return pl.pallas_call(
        _gmm_body,
        grid_spec=grid_spec,
        out_shape=jax.ShapeDtypeStruct((PMAX, D), jnp.float32),
    )(bexp, xs, ws, w13, w2)


CTC = 8
NCHC = TOK_W // CTC


def _combine_body(ys_hbm, p_hbm, out_hbm, *scr):
    bufs = (scr[0:4], scr[4:8])
    sem_ld = scr[8]
    sem_gt = scr[9]
    sem_st = scr[10]
    wid = lax.axis_index("s") * 2 + lax.axis_index("c")
    base = wid * TOK_W

    def fire_gathers(ci, bs):
        b0, b1, i0, i1 = bs
        tb = base + ci * CTC
        for d in [pltpu.async_copy(p_hbm.at[0, pl.ds(tb, CTC)], i0, sem_ld),
                  pltpu.async_copy(p_hbm.at[1, pl.ds(tb, CTC)], i1, sem_ld)]:
            d.wait()
        return [
            pltpu.async_copy(ys_hbm.at[i0], b0, sem_gt),
            pltpu.async_copy(ys_hbm.at[i1], b1, sem_gt),
        ]

    pend = fire_gathers(0, bufs[0])
    out_pend = []
    for ci in range(NCHC):
        for d in pend:
            d.wait()
        for d in out_pend:
            d.wait()
        if ci + 1 < NCHC:
            pend = fire_gathers(ci + 1, bufs[(ci + 1) % 2])
        b0, b1, _, _ = bufs[ci % 2]

        @plsc.parallel_loop(0, CTC * (D // 16), 1, unroll=16)
        def _body(i):
            j = lax.shift_right_logical(i, 7)
            c = lax.shift_left(jnp.bitwise_and(i, D // 16 - 1), 4)
            sl = pl.ds(pl.multiple_of(c, 16), 16)
            b0[j, sl] = b0[j, sl] + b1[j, sl]

        out_pend = [pltpu.async_copy(
            b0, out_hbm.at[pl.ds(base + ci * CTC, CTC)], sem_st)]
    for d in out_pend:
        d.wait()


def _combine(ys, pos):
    mesh = plsc.VectorSubcoreMesh(core_axis_name="c", subcore_axis_name="s", num_cores=2, num_subcores=16)
    bufset = [
        pltpu.VMEM((CTC, D), jnp.float32),
        pltpu.VMEM((CTC, D), jnp.float32),
        pltpu.VMEM((CTC,), jnp.int32),
        pltpu.VMEM((CTC,), jnp.int32),
    ]
    f = pl.kernel(
        _combine_body,
        out_type=jax.ShapeDtypeStruct((T, D), jnp.float32),
        mesh=mesh,
        scratch_types=bufset + bufset + [pltpu.SemaphoreType.DMA,
                                         pltpu.SemaphoreType.DMA,
                                         pltpu.SemaphoreType.DMA],
    )
    return f(ys, pos)


@jax.jit
def kernel(hidden_states, gate_weight, w13_weight, w2_weight):
    pos, sc1, sc2, bexp = _route(hidden_states, gate_weight)
    xs, ws = _dispatch(hidden_states, pos, sc1, sc2)
    ys = _gmm(bexp, xs, ws, w13_weight, w2_weight)
    return _combine(ys, pos)


# speedup vs baseline: 1.0122x; 1.0122x over previous
import functools

import jax
import jax.numpy as jnp
from jax import lax
from jax.experimental import pallas as pl
from jax.experimental.pallas import tpu as pltpu
from jax.experimental.pallas import tpu_sc as plsc

T = 2048
D = 2048
E = 8
DFF = 768
K = 2

BLK = 256
PMAX = T * K + E * BLK
NB = PMAX // BLK

NW = 32
TOK_W = T // NW
CT = 16


def _route_body(x_ref, gw_ref, pos_ref, sc1_ref, sc2_ref, bexp_ref):
    x = x_ref[...]
    logits = lax.dot_general(x, gw_ref[...], (((1,), (1,)), ((), ())),
                             preferred_element_type=jnp.float32)
    probs = jax.nn.softmax(logits, axis=-1)
    iota_e = lax.broadcasted_iota(jnp.int32, probs.shape, 1)
    m1 = jnp.max(probs, axis=1, keepdims=True)
    idx1 = jnp.min(jnp.where(probs == m1, iota_e, E), axis=1, keepdims=True)
    oh1 = iota_e == idx1
    p2 = jnp.where(oh1, -jnp.inf, probs)
    m2 = jnp.max(p2, axis=1, keepdims=True)
    idx2 = jnp.min(jnp.where(p2 == m2, iota_e, E), axis=1, keepdims=True)
    oh2 = iota_e == idx2
    denom = m1 + m2
    s1 = m1 / denom
    s2 = m2 / denom

    scnt = oh1.astype(jnp.float32) + oh2.astype(jnp.float32)
    r_i = lax.broadcasted_iota(jnp.int32, (T, T), 0)
    c_i = lax.broadcasted_iota(jnp.int32, (T, T), 1)
    ltri = (c_i < r_i).astype(jnp.float32)
    csum = lax.dot_general(ltri, scnt, (((1,), (0,)), ((), ())),
                           preferred_element_type=jnp.float32)
    counts = csum[T - 1:T, :] + scnt[T - 1:T, :]
    padded = jnp.floor((counts + (BLK - 1)) * (1.0 / BLK)).astype(jnp.float32)
    padded = padded * BLK
    a_i = lax.broadcasted_iota(jnp.int32, (E, E), 0)
    b_i = lax.broadcasted_iota(jnp.int32, (E, E), 1)
    utri = (a_i < b_i).astype(jnp.float32)
    offs = lax.dot_general(padded, utri, (((1,), (0,)), ((), ())),
                           preferred_element_type=jnp.float32)

    slot = offs + csum
    pos1 = jnp.sum(jnp.where(oh1, slot, 0.0), axis=1, keepdims=True)
    pos2 = jnp.sum(jnp.where(oh2, slot, 0.0), axis=1, keepdims=True)

    rowp = lax.broadcasted_iota(jnp.int32, (8, T), 0)
    pos1_t = jnp.transpose(pos1.astype(jnp.int32))
    pos2_t = jnp.transpose(pos2.astype(jnp.int32))
    pos_ref[...] = jnp.where(rowp == 0, pos1_t,
                             jnp.where(rowp == 1, pos2_t, 0))
    sc1_ref[...] = jnp.broadcast_to(s1, (T, 128))
    sc2_ref[...] = jnp.broadcast_to(s2, (T, 128))

    rowb = lax.broadcasted_iota(jnp.int32, (8, 128), 0)
    laneb = lax.broadcasted_iota(jnp.int32, (8, 128), 1)
    startf = (laneb * BLK).astype(jnp.float32)
    ends = offs + padded
    acc = jnp.zeros((8, 128), jnp.int32)
    for e in range(E - 1):
        acc = acc + (startf >= ends[0:1, e:e + 1]).astype(jnp.int32)
    total = ends[0:1, E - 1:E]
    valid = (startf < total).astype(jnp.int32)
    bexp_ref[...] = jnp.where(rowb == 0, acc, valid)


def _route(x, gw):
    return pl.pallas_call(
        _route_body,
        out_shape=(
            jax.ShapeDtypeStruct((8, T), jnp.int32),
            jax.ShapeDtypeStruct((T, 128), jnp.float32),
            jax.ShapeDtypeStruct((T, 128), jnp.float32),
            jax.ShapeDtypeStruct((8, 128), jnp.int32),
        ),
    )(x, gw)


NCH = TOK_W // CT


def _dispatch_body(x_hbm, p_hbm, s1_hbm, s2_hbm, xs_hbm, ws_hbm, *scr):
    bufs = (scr[0:5], scr[5:10])
    sem_ld = scr[10]
    sem_st = scr[11]
    wid = lax.axis_index("s") * 2 + lax.axis_index("c")
    base = wid * TOK_W

    def fire_loads(ci, bs):
        xbuf, i0, i1, sv0, sv1 = bs
        tb = base + ci * CT
        return [
            pltpu.async_copy(x_hbm.at[pl.ds(tb, CT)], xbuf, sem_ld),
            pltpu.async_copy(p_hbm.at[0, pl.ds(tb, CT)], i0, sem_ld),
            pltpu.async_copy(p_hbm.at[1, pl.ds(tb, CT)], i1, sem_ld),
            pltpu.async_copy(s1_hbm.at[pl.ds(tb, CT)], sv0, sem_ld),
            pltpu.async_copy(s2_hbm.at[pl.ds(tb, CT)], sv1, sem_ld),
        ]

    def fire_scats(bs):
        xbuf, i0, i1, sv0, sv1 = bs
        return [
            pltpu.async_copy(xbuf, xs_hbm.at[i0], sem_st),
            pltpu.async_copy(xbuf, xs_hbm.at[i1], sem_st),
            pltpu.async_copy(sv0, ws_hbm.at[i0], sem_st),
            pltpu.async_copy(sv1, ws_hbm.at[i1], sem_st),
        ]

    lds = fire_loads(0, bufs[0])
    pend = []
    for ci in range(NCH):
        for d in lds:
            d.wait()
        for d in pend:
            d.wait()
        if ci + 1 < NCH:
            lds = fire_loads(ci + 1, bufs[(ci + 1) % 2])
        pend = fire_scats(bufs[ci % 2])
    for d in pend:
        d.wait()


def _dispatch(x, pos, s1, s2):
    mesh = plsc.VectorSubcoreMesh(core_axis_name="c", subcore_axis_name="s", num_cores=2, num_subcores=16)
    bufset = [
        pltpu.VMEM((CT, D), jnp.float32),
        pltpu.VMEM((CT,), jnp.int32),
        pltpu.VMEM((CT,), jnp.int32),
        pltpu.VMEM((CT, 128), jnp.float32),
        pltpu.VMEM((CT, 128), jnp.float32),
    ]
    f = pl.kernel(
        _dispatch_body,
        out_type=(
            jax.ShapeDtypeStruct((PMAX, D), jnp.float32),
            jax.ShapeDtypeStruct((PMAX, 128), jnp.float32),
        ),
        mesh=mesh,
        scratch_types=bufset + bufset + [pltpu.SemaphoreType.DMA,
                                         pltpu.SemaphoreType.DMA],
    )
    return f(x, pos, s1, s2)


def _gmm_body(bexp_ref, xs_ref, ws_ref, w13_ref, w2_ref, ys_ref):
    @pl.when(bexp_ref[1, pl.program_id(0)] == 1)
    def _():
        xs = xs_ref[...]
        g = lax.dot_general(xs, w13_ref[0, :DFF, :], (((1,), (1,)), ((), ())),
                            preferred_element_type=jnp.float32)
        u = lax.dot_general(xs, w13_ref[0, DFF:, :], (((1,), (1,)), ((), ())),
                            preferred_element_type=jnp.float32)
        act = g * jax.nn.sigmoid(g) * u
        y = lax.dot_general(act, w2_ref[0], (((1,), (1,)), ((), ())),
                            preferred_element_type=jnp.float32)
        ys_ref[...] = y * ws_ref[:, 0:1]


def _gmm(bexp, xs, ws, w13, w2):
    grid_spec = pltpu.PrefetchScalarGridSpec(
        num_scalar_prefetch=1,
        grid=(NB,),
        in_specs=[
            pl.BlockSpec((BLK, D), lambda b, be: (be[1, b] * b, 0)),
            pl.BlockSpec((BLK, 128), lambda b, be: (be[1, b] * b, 0)),
            pl.BlockSpec((1, 2 * DFF, D), lambda b, be: (be[0, b], 0, 0)),
            pl.BlockSpec((1, D, DFF), lambda b, be: (be[0, b], 0, 0)),
        ],
        out_specs=pl.BlockSpec((BLK, D), lambda b, be: (b, 0)),
    )
    return pl.pallas_call(
        _gmm_body,
        grid_spec=grid_spec,
        out_shape=jax.ShapeDtypeStruct((PMAX, D), jnp.float32),
    )(bexp, xs, ws, w13, w2)


CTC = 8
NCHC = TOK_W // CTC


def _combine_body(ys_hbm, p_hbm, out_hbm, *scr):
    bufs = (scr[0:4], scr[4:8])
    sem_ld = scr[8]
    sem_gt = scr[9]
    sem_st = scr[10]
    wid = lax.axis_index("s") * 2 + lax.axis_index("c")
    base = wid * TOK_W

    def fire_gathers(ci, bs):
        b0, b1, i0, i1 = bs
        tb = base + ci * CTC
        for d in [pltpu.async_copy(p_hbm.at[0, pl.ds(tb, CTC)], i0, sem_ld),
                  pltpu.async_copy(p_hbm.at[1, pl.ds(tb, CTC)], i1, sem_ld)]:
            d.wait()
        return [
            pltpu.async_copy(ys_hbm.at[i0], b0, sem_gt),
            pltpu.async_copy(ys_hbm.at[i1], b1, sem_gt),
        ]

    pend = fire_gathers(0, bufs[0])
    out_pend = []
    for ci in range(NCHC):
        for d in pend:
            d.wait()
        for d in out_pend:
            d.wait()
        if ci + 1 < NCHC:
            pend = fire_gathers(ci + 1, bufs[(ci + 1) % 2])
        b0, b1, _, _ = bufs[ci % 2]

        @plsc.parallel_loop(0, CTC * (D // 16), 1, unroll=16)
        def _body(i):
            j = lax.shift_right_logical(i, 7)
            c = lax.shift_left(jnp.bitwise_and(i, D // 16 - 1), 4)
            sl = pl.ds(pl.multiple_of(c, 16), 16)
            b0[j, sl] = b0[j, sl] + b1[j, sl]

        out_pend = [pltpu.async_copy(
            b0, out_hbm.at[pl.ds(base + ci * CTC, CTC)], sem_st)]
    for d in out_pend:
        d.wait()


def _combine(ys, pos):
    mesh = plsc.VectorSubcoreMesh(core_axis_name="c", subcore_axis_name="s", num_cores=2, num_subcores=16)
    bufset = [
        pltpu.VMEM((CTC, D), jnp.float32),
        pltpu.VMEM((CTC, D), jnp.float32),
        pltpu.VMEM((CTC,), jnp.int32),
        pltpu.VMEM((CTC,), jnp.int32),
    ]
    f = pl.kernel(
        _combine_body,
        out_type=jax.ShapeDtypeStruct((T, D), jnp.float32),
        mesh=mesh,
        scratch_types=bufset + bufset + [pltpu.SemaphoreType.DMA,
                                         pltpu.SemaphoreType.DMA,
                                         pltpu.SemaphoreType.DMA],
    )
    return f(ys, pos)


@jax.jit
def kernel(hidden_states, gate_weight, w13_weight, w2_weight):
    pos, sc1, sc2, bexp = _route(hidden_states, gate_weight)
    xs, ws = _dispatch(hidden_states, pos, sc1, sc2)
    ys = _gmm(bexp, xs, ws, w13_weight, w2_weight)
    return _combine(ys, pos)
